# Initial kernel scaffold; baseline (speedup 1.0000x reference)
import jax
import jax.numpy as jnp
from jax.experimental import pallas as pl


def kernel(x, edge_index, batch, W1, b1, W2, b2, fcW1, fcb1, fcW2, fcb2):
    raise NotImplementedError("write your pallas kernel here")


# speedup vs baseline: 18.3691x; 18.3691x over previous
import functools

import jax
import jax.numpy as jnp
from jax import lax
from jax.experimental import pallas as pl
from jax.experimental.pallas import tpu as pltpu
from jax.experimental.pallas import tpu_sc as plsc

N_NODES = 10000
N_EDGES = 320000
D_FEAT = 128
HIDDEN = 32
N_CLASSES = 10
N_GRAPHS = 64

NC = 2
NS = 16
NW = NC * NS

EDGES_PER_TILE = N_EDGES // NW
CHUNK = 128
K_CHUNKS = (EDGES_PER_TILE + CHUNK - 1) // CHUNK
EDGES_PER_TILE_PAD = K_CHUNKS * CHUNK
ROWS_PER_TILE = 632
N_ACC = ROWS_PER_TILE * NS
DUMMY_ROW = N_NODES

ROW_BLK = 1000
N_BLKS = N_NODES // ROW_BLK


def _sc_agg_kernel(table_hbm, src_hbm, dst_hbm, out_hbm,
                   src_v, dst_v, rows_v, zbuf_v, acc_sh, sem):
  cid = lax.axis_index("c")
  sid = lax.axis_index("s")
  wid = sid * NC + cid

  def zero_row(r, _):
    zbuf_v[r, pl.ds(0, 16)] = jnp.zeros((16,), jnp.float32)
    zbuf_v[r, pl.ds(16, 16)] = jnp.zeros((16,), jnp.float32)
    return 0
  lax.fori_loop(0, ROWS_PER_TILE, zero_row, 0)
  pltpu.sync_copy(zbuf_v, acc_sh.at[pl.ds(sid * ROWS_PER_TILE, ROWS_PER_TILE)])

  pltpu.sync_copy(src_hbm.at[wid], src_v)
  pltpu.sync_copy(dst_hbm.at[wid], dst_v)
  plsc.subcore_barrier()

  def body(j, _):
    pltpu.async_copy(table_hbm.at[src_v.at[j]], rows_v, sem).wait()
    pltpu.sync_copy(rows_v, acc_sh.at[dst_v.at[j]], add=True)
    return 0
  lax.fori_loop(0, K_CHUNKS, body, 0)

  plsc.subcore_barrier()
  pltpu.sync_copy(acc_sh.at[pl.ds(sid * ROWS_PER_TILE, ROWS_PER_TILE)],
                  zbuf_v)
  pltpu.sync_copy(zbuf_v,
                  out_hbm.at[cid, pl.ds(sid * ROWS_PER_TILE, ROWS_PER_TILE)])


@functools.cache
def _get_sc_agg():
  return pl.kernel(
      _sc_agg_kernel,
      out_type=jax.ShapeDtypeStruct((NC, N_ACC, HIDDEN), jnp.float32),
      mesh=plsc.VectorSubcoreMesh(core_axis_name="c", subcore_axis_name="s",
                                  num_cores=NC, num_subcores=NS),
      compiler_params=pltpu.CompilerParams(use_tc_tiling_on_sc=False),
      scratch_types=[
          pltpu.VMEM((K_CHUNKS, CHUNK), jnp.int32),
          pltpu.VMEM((K_CHUNKS, CHUNK), jnp.int32),
          pltpu.VMEM((CHUNK, HIDDEN), jnp.float32),
          pltpu.VMEM((ROWS_PER_TILE, HIDDEN), jnp.float32),
          pltpu.VMEM_SHARED((N_ACC, HIDDEN), jnp.float32),
          pltpu.SemaphoreType.DMA,
      ],
  )


def _tc_a_body(x_ref, w1_ref, degp_ref, hs_ref):
  xr = x_ref[...]
  s = jnp.sum(xr, axis=1, keepdims=True)
  xn = xr / jnp.maximum(s, 1.0)
  h = jnp.dot(xn, w1_ref[...], preferred_element_type=jnp.float32)
  deg = degp_ref[0, :, 0:1] + degp_ref[1, :, 0:1] + 1.0
  dinv = lax.rsqrt(deg)
  hs_ref[...] = h * dinv


def _tc_b_body(aggp_ref, hs1_ref, degp_ref, b1_ref, w2_ref, hs2_ref):
  deg = degp_ref[0, :, 0:1] + degp_ref[1, :, 0:1] + 1.0
  dinv = lax.rsqrt(deg)
  t = dinv * (aggp_ref[0] + aggp_ref[1] + hs1_ref[...]) + b1_ref[...]
  t = jnp.maximum(t, 0.0)
  h2 = jnp.dot(t, w2_ref[...], preferred_element_type=jnp.float32)
  hs2_ref[...] = h2 * dinv


def _tc_c_body(aggp_ref, hs2_ref, degp_ref, b2_ref, batch_ref,
               fcw1_ref, fcb1_ref, fcw2_ref, fcb2_ref, out_ref, acc_ref):
  i = pl.program_id(0)

  @pl.when(i == 0)
  def _():
    acc_ref[...] = jnp.zeros((N_GRAPHS, HIDDEN), jnp.float32)

  deg = degp_ref[0, :, 0:1] + degp_ref[1, :, 0:1] + 1.0
  dinv = lax.rsqrt(deg)
  h2f = dinv * (aggp_ref[0] + aggp_ref[1] + hs2_ref[...]) + b2_ref[...]
  b = batch_ref[0, 0, :]
  onehot = (b[:, None] == lax.broadcasted_iota(jnp.int32, (1, N_GRAPHS), 1)
            ).astype(jnp.float32)
  gp = lax.dot_general(onehot, h2f, (((0,), (0,)), ((), ())),
                       preferred_element_type=jnp.float32)
  acc = acc_ref[...] + gp
  acc_ref[...] = acc

  @pl.when(i == N_BLKS - 1)
  def _():
    g1 = jnp.maximum(
        jnp.dot(acc, fcw1_ref[...], preferred_element_type=jnp.float32)
        + fcb1_ref[...], 0.0)
    logits = jnp.dot(g1, fcw2_ref[...],
                     preferred_element_type=jnp.float32) + fcb2_ref[...]
    m = jnp.max(logits, axis=1, keepdims=True)
    z = logits - m
    lse = jnp.log(jnp.sum(jnp.exp(z), axis=1, keepdims=True))
    out_ref[...] = z - lse


def _row_blk(i):
  return (i, 0)


def _full(i):
  return (0, 0)


def _part_blk(i):
  return (0, i, 0)


_tc_a = pl.pallas_call(
    _tc_a_body,
    grid=(N_BLKS,),
    in_specs=[
        pl.BlockSpec((ROW_BLK, D_FEAT), _row_blk),
        pl.BlockSpec((D_FEAT, HIDDEN), _full),
        pl.BlockSpec((2, ROW_BLK, HIDDEN), _part_blk),
    ],
    out_specs=pl.BlockSpec((ROW_BLK, HIDDEN), _row_blk),
    out_shape=jax.ShapeDtypeStruct((N_NODES, HIDDEN), jnp.float32),
)

_tc_b = pl.pallas_call(
    _tc_b_body,
    grid=(N_BLKS,),
    in_specs=[
        pl.BlockSpec((2, ROW_BLK, HIDDEN), _part_blk),
        pl.BlockSpec((ROW_BLK, HIDDEN), _row_blk),
        pl.BlockSpec((2, ROW_BLK, HIDDEN), _part_blk),
        pl.BlockSpec((1, HIDDEN), _full),
        pl.BlockSpec((HIDDEN, HIDDEN), _full),
    ],
    out_specs=pl.BlockSpec((ROW_BLK, HIDDEN), _row_blk),
    out_shape=jax.ShapeDtypeStruct((N_NODES, HIDDEN), jnp.float32),
)

_tc_c = pl.pallas_call(
    _tc_c_body,
    grid=(N_BLKS,),
    in_specs=[
        pl.BlockSpec((2, ROW_BLK, HIDDEN), _part_blk),
        pl.BlockSpec((ROW_BLK, HIDDEN), _row_blk),
        pl.BlockSpec((2, ROW_BLK, HIDDEN), _part_blk),
        pl.BlockSpec((1, HIDDEN), _full),
        pl.BlockSpec((1, 1, ROW_BLK), lambda i: (i, 0, 0)),
        pl.BlockSpec((HIDDEN, HIDDEN), _full),
        pl.BlockSpec((1, HIDDEN), _full),
        pl.BlockSpec((HIDDEN, N_CLASSES), _full),
        pl.BlockSpec((1, N_CLASSES), _full),
    ],
    out_specs=pl.BlockSpec((N_GRAPHS, N_CLASSES), _full),
    out_shape=jax.ShapeDtypeStruct((N_GRAPHS, N_CLASSES), jnp.float32),
    scratch_shapes=[pltpu.VMEM((N_GRAPHS, HIDDEN), jnp.float32)],
)


@jax.jit
def kernel(x, edge_index, batch, W1, b1, W2, b2, fcW1, fcb1, fcW2, fcb2):
  src = edge_index[0].astype(jnp.int32)
  dst = edge_index[1].astype(jnp.int32)

  pad = EDGES_PER_TILE_PAD - EDGES_PER_TILE
  srcp = jnp.concatenate(
      [src.reshape(NW, EDGES_PER_TILE),
       jnp.zeros((NW, pad), jnp.int32)], axis=1).reshape(NW, K_CHUNKS, CHUNK)
  dstp = jnp.concatenate(
      [dst.reshape(NW, EDGES_PER_TILE),
       jnp.full((NW, pad), DUMMY_ROW, jnp.int32)],
      axis=1).reshape(NW, K_CHUNKS, CHUNK)

  sc_agg = _get_sc_agg()
  ones_tbl = jnp.ones((N_NODES, HIDDEN), jnp.float32)
  degp = sc_agg(ones_tbl, srcp, dstp)[:, :N_NODES, :]

  hs1 = _tc_a(x, W1, degp)
  agg1 = sc_agg(hs1, srcp, dstp)[:, :N_NODES, :]
  hs2 = _tc_b(agg1, hs1, degp, b1.reshape(1, HIDDEN), W2)
  agg2 = sc_agg(hs2, srcp, dstp)[:, :N_NODES, :]
  out = _tc_c(agg2, hs2, degp, b2.reshape(1, HIDDEN),
              batch.astype(jnp.int32).reshape(N_BLKS, 1, ROW_BLK),
              fcW1, fcb1.reshape(1, HIDDEN), fcW2,
              fcb2.reshape(1, N_CLASSES))
  return out


# speedup vs baseline: 24.3144x; 1.3237x over previous
import functools

import jax
import jax.numpy as jnp
from jax import lax
from jax.experimental import pallas as pl
from jax.experimental.pallas import tpu as pltpu
from jax.experimental.pallas import tpu_sc as plsc

N_NODES = 10000
N_EDGES = 320000
D_FEAT = 128
HIDDEN = 32
N_CLASSES = 10
N_GRAPHS = 64

NC = 2
NS = 16
NW = NC * NS

EDGES_PER_TILE = N_EDGES // NW
CHUNK = 128
NBUF = 8
K_CHUNKS = 80
EDGES_PER_TILE_PAD = K_CHUNKS * CHUNK
DEG_W = 16
ROWS_PER_TILE = 632
N_ACC = ROWS_PER_TILE * NS
DUMMY_ROW = N_NODES

ROW_BLK = 1000
N_BLKS = N_NODES // ROW_BLK


def _zero_stripe(zbuf_v, acc_sh, sid, width):
  def zero_row(r, _):
    for c0 in range(0, width, 16):
      zbuf_v[r, pl.ds(c0, 16)] = jnp.zeros((16,), jnp.float32)
    return 0
  lax.fori_loop(0, ROWS_PER_TILE, zero_row, 0)
  pltpu.sync_copy(zbuf_v, acc_sh.at[pl.ds(sid * ROWS_PER_TILE, ROWS_PER_TILE)])


def _writeout(zbuf_v, acc_sh, out_hbm, cid, sid):
  pltpu.sync_copy(acc_sh.at[pl.ds(sid * ROWS_PER_TILE, ROWS_PER_TILE)],
                  zbuf_v)
  pltpu.sync_copy(zbuf_v,
                  out_hbm.at[cid, pl.ds(sid * ROWS_PER_TILE, ROWS_PER_TILE)])


def _sc_agg_kernel(table_hbm, src_hbm, dst_hbm, out_hbm,
                   src_v, dst_v, rows_v, zbuf_v, acc_sh, sem_g, sem_s):
  cid = lax.axis_index("c")
  sid = lax.axis_index("s")
  wid = sid * NC + cid

  _zero_stripe(zbuf_v, acc_sh, sid, HIDDEN)
  pltpu.sync_copy(src_hbm.at[wid], src_v)
  pltpu.sync_copy(dst_hbm.at[wid], dst_v)
  plsc.subcore_barrier()

  def group(g, _):
    j0 = g * NBUF
    gd = [pltpu.async_copy(table_hbm.at[src_v.at[j0 + b]], rows_v.at[b],
                           sem_g) for b in range(NBUF)]
    sd = []
    for b in range(NBUF):
      gd[b].wait()
      sd.append(pltpu.async_copy(rows_v.at[b], acc_sh.at[dst_v.at[j0 + b]],
                                 sem_s, add=True))
    for d in sd:
      d.wait()
    return 0
  lax.fori_loop(0, K_CHUNKS // NBUF, group, 0)

  plsc.subcore_barrier()
  _writeout(zbuf_v, acc_sh, out_hbm, cid, sid)


def _sc_deg_kernel(dst_hbm, out_hbm, dst_v, ones_v, zbuf_v, acc_sh, sem_s):
  cid = lax.axis_index("c")
  sid = lax.axis_index("s")
  wid = sid * NC + cid

  def one_row(r, _):
    ones_v[r, pl.ds(0, 16)] = jnp.ones((16,), jnp.float32)
    return 0
  lax.fori_loop(0, CHUNK, one_row, 0)
  _zero_stripe(zbuf_v, acc_sh, sid, DEG_W)
  pltpu.sync_copy(dst_hbm.at[wid], dst_v)
  plsc.subcore_barrier()

  def group(g, _):
    j0 = g * NBUF
    sd = [pltpu.async_copy(ones_v, acc_sh.at[dst_v.at[j0 + b]],
                           sem_s, add=True) for b in range(NBUF)]
    for d in sd:
      d.wait()
    return 0
  lax.fori_loop(0, K_CHUNKS // NBUF, group, 0)

  plsc.subcore_barrier()
  _writeout(zbuf_v, acc_sh, out_hbm, cid, sid)


@functools.cache
def _get_sc_agg():
  return pl.kernel(
      _sc_agg_kernel,
      out_type=jax.ShapeDtypeStruct((NC, N_ACC, HIDDEN), jnp.float32),
      mesh=plsc.VectorSubcoreMesh(core_axis_name="c", subcore_axis_name="s",
                                  num_cores=NC, num_subcores=NS),
      compiler_params=pltpu.CompilerParams(use_tc_tiling_on_sc=False),
      scratch_types=[
          pltpu.VMEM((K_CHUNKS, CHUNK), jnp.int32),
          pltpu.VMEM((K_CHUNKS, CHUNK), jnp.int32),
          pltpu.VMEM((NBUF, CHUNK, HIDDEN), jnp.float32),
          pltpu.VMEM((ROWS_PER_TILE, HIDDEN), jnp.float32),
          pltpu.VMEM_SHARED((N_ACC, HIDDEN), jnp.float32),
          pltpu.SemaphoreType.DMA,
          pltpu.SemaphoreType.DMA,
      ],
  )


@functools.cache
def _get_sc_deg():
  return pl.kernel(
      _sc_deg_kernel,
      out_type=jax.ShapeDtypeStruct((NC, N_ACC, DEG_W), jnp.float32),
      mesh=plsc.VectorSubcoreMesh(core_axis_name="c", subcore_axis_name="s",
                                  num_cores=NC, num_subcores=NS),
      compiler_params=pltpu.CompilerParams(use_tc_tiling_on_sc=False),
      scratch_types=[
          pltpu.VMEM((K_CHUNKS, CHUNK), jnp.int32),
          pltpu.VMEM((CHUNK, DEG_W), jnp.float32),
          pltpu.VMEM((ROWS_PER_TILE, DEG_W), jnp.float32),
          pltpu.VMEM_SHARED((N_ACC, DEG_W), jnp.float32),
          pltpu.SemaphoreType.DMA,
      ],
  )


def _tc_a_body(x_ref, w1_ref, degp_ref, hs_ref):
  xr = x_ref[...]
  s = jnp.sum(xr, axis=1, keepdims=True)
  xn = xr / jnp.maximum(s, 1.0)
  h = jnp.dot(xn, w1_ref[...], preferred_element_type=jnp.float32)
  deg = degp_ref[0, :, 0:1] + degp_ref[1, :, 0:1] + 1.0
  dinv = lax.rsqrt(deg)
  hs_ref[...] = h * dinv


def _tc_b_body(aggp_ref, hs1_ref, degp_ref, b1_ref, w2_ref, hs2_ref):
  deg = degp_ref[0, :, 0:1] + degp_ref[1, :, 0:1] + 1.0
  dinv = lax.rsqrt(deg)
  t = dinv * (aggp_ref[0] + aggp_ref[1] + hs1_ref[...]) + b1_ref[...]
  t = jnp.maximum(t, 0.0)
  h2 = jnp.dot(t, w2_ref[...], preferred_element_type=jnp.float32)
  hs2_ref[...] = h2 * dinv


def _tc_c_body(aggp_ref, hs2_ref, degp_ref, b2_ref, batch_ref,
               fcw1_ref, fcb1_ref, fcw2_ref, fcb2_ref, out_ref, acc_ref):
  i = pl.program_id(0)

  @pl.when(i == 0)
  def _():
    acc_ref[...] = jnp.zeros((N_GRAPHS, HIDDEN), jnp.float32)

  deg = degp_ref[0, :, 0:1] + degp_ref[1, :, 0:1] + 1.0
  dinv = lax.rsqrt(deg)
  h2f = dinv * (aggp_ref[0] + aggp_ref[1] + hs2_ref[...]) + b2_ref[...]
  b = batch_ref[0, 0, :]
  onehot = (b[:, None] == lax.broadcasted_iota(jnp.int32, (1, N_GRAPHS), 1)
            ).astype(jnp.float32)
  gp = lax.dot_general(onehot, h2f, (((0,), (0,)), ((), ())),
                       preferred_element_type=jnp.float32)
  acc = acc_ref[...] + gp
  acc_ref[...] = acc

  @pl.when(i == N_BLKS - 1)
  def _():
    g1 = jnp.maximum(
        jnp.dot(acc, fcw1_ref[...], preferred_element_type=jnp.float32)
        + fcb1_ref[...], 0.0)
    logits = jnp.dot(g1, fcw2_ref[...],
                     preferred_element_type=jnp.float32) + fcb2_ref[...]
    m = jnp.max(logits, axis=1, keepdims=True)
    z = logits - m
    lse = jnp.log(jnp.sum(jnp.exp(z), axis=1, keepdims=True))
    out_ref[...] = z - lse


def _row_blk(i):
  return (i, 0)


def _full(i):
  return (0, 0)


def _part_blk(i):
  return (0, i, 0)


_tc_a = pl.pallas_call(
    _tc_a_body,
    grid=(N_BLKS,),
    in_specs=[
        pl.BlockSpec((ROW_BLK, D_FEAT), _row_blk),
        pl.BlockSpec((D_FEAT, HIDDEN), _full),
        pl.BlockSpec((2, ROW_BLK, DEG_W), _part_blk),
    ],
    out_specs=pl.BlockSpec((ROW_BLK, HIDDEN), _row_blk),
    out_shape=jax.ShapeDtypeStruct((N_NODES, HIDDEN), jnp.float32),
)

_tc_b = pl.pallas_call(
    _tc_b_body,
    grid=(N_BLKS,),
    in_specs=[
        pl.BlockSpec((2, ROW_BLK, HIDDEN), _part_blk),
        pl.BlockSpec((ROW_BLK, HIDDEN), _row_blk),
        pl.BlockSpec((2, ROW_BLK, DEG_W), _part_blk),
        pl.BlockSpec((1, HIDDEN), _full),
        pl.BlockSpec((HIDDEN, HIDDEN), _full),
    ],
    out_specs=pl.BlockSpec((ROW_BLK, HIDDEN), _row_blk),
    out_shape=jax.ShapeDtypeStruct((N_NODES, HIDDEN), jnp.float32),
)

_tc_c = pl.pallas_call(
    _tc_c_body,
    grid=(N_BLKS,),
    in_specs=[
        pl.BlockSpec((2, ROW_BLK, HIDDEN), _part_blk),
        pl.BlockSpec((ROW_BLK, HIDDEN), _row_blk),
        pl.BlockSpec((2, ROW_BLK, DEG_W), _part_blk),
        pl.BlockSpec((1, HIDDEN), _full),
        pl.BlockSpec((1, 1, ROW_BLK), lambda i: (i, 0, 0)),
        pl.BlockSpec((HIDDEN, HIDDEN), _full),
        pl.BlockSpec((1, HIDDEN), _full),
        pl.BlockSpec((HIDDEN, N_CLASSES), _full),
        pl.BlockSpec((1, N_CLASSES), _full),
    ],
    out_specs=pl.BlockSpec((N_GRAPHS, N_CLASSES), _full),
    out_shape=jax.ShapeDtypeStruct((N_GRAPHS, N_CLASSES), jnp.float32),
    scratch_shapes=[pltpu.VMEM((N_GRAPHS, HIDDEN), jnp.float32)],
)


@jax.jit
def kernel(x, edge_index, batch, W1, b1, W2, b2, fcW1, fcb1, fcW2, fcb2):
  src = edge_index[0].astype(jnp.int32)
  dst = edge_index[1].astype(jnp.int32)

  pad = EDGES_PER_TILE_PAD - EDGES_PER_TILE
  srcp = jnp.concatenate(
      [src.reshape(NW, EDGES_PER_TILE),
       jnp.zeros((NW, pad), jnp.int32)], axis=1).reshape(NW, K_CHUNKS, CHUNK)
  dstp = jnp.concatenate(
      [dst.reshape(NW, EDGES_PER_TILE),
       jnp.full((NW, pad), DUMMY_ROW, jnp.int32)],
      axis=1).reshape(NW, K_CHUNKS, CHUNK)

  sc_agg = _get_sc_agg()
  degp = _get_sc_deg()(dstp)[:, :N_NODES, :]

  hs1 = _tc_a(x, W1, degp)
  agg1 = sc_agg(hs1, srcp, dstp)[:, :N_NODES, :]
  hs2 = _tc_b(agg1, hs1, degp, b1.reshape(1, HIDDEN), W2)
  agg2 = sc_agg(hs2, srcp, dstp)[:, :N_NODES, :]
  out = _tc_c(agg2, hs2, degp, b2.reshape(1, HIDDEN),
              batch.astype(jnp.int32).reshape(N_BLKS, 1, ROW_BLK),
              fcW1, fcb1.reshape(1, HIDDEN), fcW2,
              fcb2.reshape(1, N_CLASSES))
  return out


# speedup vs baseline: 26.1151x; 1.0741x over previous
import functools

import jax
import jax.numpy as jnp
from jax import lax
from jax.experimental import pallas as pl
from jax.experimental.pallas import tpu as pltpu
from jax.experimental.pallas import tpu_sc as plsc

N_NODES = 10000
N_EDGES = 320000
D_FEAT = 128
HIDDEN = 32
N_CLASSES = 10
N_GRAPHS = 64

NC = 2
NS = 16
NW = NC * NS

EDGES_PER_TILE = N_EDGES // NW
CHUNK = 128
NBUF = 8
K_CHUNKS = 80
EDGES_PER_TILE_PAD = K_CHUNKS * CHUNK
DEG_W = 16
ROWS_PER_TILE = 632
N_ACC = ROWS_PER_TILE * NS
DUMMY_ROW = N_NODES

ROW_BLK = 1000
N_BLKS = N_NODES // ROW_BLK


def _zero_stripe(zbuf_v, acc_sh, sid, width):
  def zero_row(r, _):
    for c0 in range(0, width, 16):
      zbuf_v[r, pl.ds(c0, 16)] = jnp.zeros((16,), jnp.float32)
    return 0
  lax.fori_loop(0, ROWS_PER_TILE, zero_row, 0)
  pltpu.sync_copy(zbuf_v, acc_sh.at[pl.ds(sid * ROWS_PER_TILE, ROWS_PER_TILE)])


def _writeout(zbuf_v, acc_sh, out_hbm, cid, sid):
  pltpu.sync_copy(acc_sh.at[pl.ds(sid * ROWS_PER_TILE, ROWS_PER_TILE)],
                  zbuf_v)
  pltpu.sync_copy(zbuf_v,
                  out_hbm.at[cid, pl.ds(sid * ROWS_PER_TILE, ROWS_PER_TILE)])


def _sc_agg_kernel(table_hbm, src_hbm, dst_hbm, out_hbm,
                   src_v, dst_v, rows_v, zbuf_v, acc_sh, sem_g, sem_s):
  cid = lax.axis_index("c")
  sid = lax.axis_index("s")
  wid = sid * NC + cid

  _zero_stripe(zbuf_v, acc_sh, sid, HIDDEN)
  pltpu.sync_copy(src_hbm.at[wid], src_v)
  pltpu.sync_copy(dst_hbm.at[wid], dst_v)
  plsc.subcore_barrier()

  def group(g, _):
    j0 = g * NBUF
    gd = []
    for b in range(NBUF):
      @pl.when(g > 0)
      def _(b=b):
        pltpu.make_async_copy(rows_v.at[b], acc_sh.at[dst_v.at[j0 + b]],
                              sem_s).wait()
      gd.append(pltpu.async_copy(table_hbm.at[src_v.at[j0 + b]],
                                 rows_v.at[b], sem_g))
    for b in range(NBUF):
      gd[b].wait()
      pltpu.async_copy(rows_v.at[b], acc_sh.at[dst_v.at[j0 + b]],
                       sem_s, add=True)
    return 0
  lax.fori_loop(0, K_CHUNKS // NBUF, group, 0)

  for b in range(NBUF):
    pltpu.make_async_copy(rows_v.at[b], acc_sh.at[dst_v.at[b]], sem_s).wait()

  plsc.subcore_barrier()
  _writeout(zbuf_v, acc_sh, out_hbm, cid, sid)


def _sc_deg_kernel(dst_hbm, out_hbm, dst_v, ones_v, zbuf_v, acc_sh, sem_s):
  cid = lax.axis_index("c")
  sid = lax.axis_index("s")
  wid = sid * NC + cid

  def one_row(r, _):
    ones_v[r, pl.ds(0, 16)] = jnp.ones((16,), jnp.float32)
    return 0
  lax.fori_loop(0, CHUNK, one_row, 0)
  _zero_stripe(zbuf_v, acc_sh, sid, DEG_W)
  pltpu.sync_copy(dst_hbm.at[wid], dst_v)
  plsc.subcore_barrier()

  def group(g, _):
    j0 = g * NBUF
    sd = [pltpu.async_copy(ones_v, acc_sh.at[dst_v.at[j0 + b]],
                           sem_s, add=True) for b in range(NBUF)]
    for d in sd:
      d.wait()
    return 0
  lax.fori_loop(0, K_CHUNKS // NBUF, group, 0)

  plsc.subcore_barrier()
  _writeout(zbuf_v, acc_sh, out_hbm, cid, sid)


@functools.cache
def _get_sc_agg():
  return pl.kernel(
      _sc_agg_kernel,
      out_type=jax.ShapeDtypeStruct((NC, N_ACC, HIDDEN), jnp.float32),
      mesh=plsc.VectorSubcoreMesh(core_axis_name="c", subcore_axis_name="s",
                                  num_cores=NC, num_subcores=NS),
      compiler_params=pltpu.CompilerParams(use_tc_tiling_on_sc=False),
      scratch_types=[
          pltpu.VMEM((K_CHUNKS, CHUNK), jnp.int32),
          pltpu.VMEM((K_CHUNKS, CHUNK), jnp.int32),
          pltpu.VMEM((NBUF, CHUNK, HIDDEN), jnp.float32),
          pltpu.VMEM((ROWS_PER_TILE, HIDDEN), jnp.float32),
          pltpu.VMEM_SHARED((N_ACC, HIDDEN), jnp.float32),
          pltpu.SemaphoreType.DMA,
          pltpu.SemaphoreType.DMA,
      ],
  )


@functools.cache
def _get_sc_deg():
  return pl.kernel(
      _sc_deg_kernel,
      out_type=jax.ShapeDtypeStruct((NC, N_ACC, DEG_W), jnp.float32),
      mesh=plsc.VectorSubcoreMesh(core_axis_name="c", subcore_axis_name="s",
                                  num_cores=NC, num_subcores=NS),
      compiler_params=pltpu.CompilerParams(use_tc_tiling_on_sc=False),
      scratch_types=[
          pltpu.VMEM((K_CHUNKS, CHUNK), jnp.int32),
          pltpu.VMEM((CHUNK, DEG_W), jnp.float32),
          pltpu.VMEM((ROWS_PER_TILE, DEG_W), jnp.float32),
          pltpu.VMEM_SHARED((N_ACC, DEG_W), jnp.float32),
          pltpu.SemaphoreType.DMA,
      ],
  )


def _tc_a_body(x_ref, w1_ref, degp_ref, hs_ref):
  xr = x_ref[...]
  s = jnp.sum(xr, axis=1, keepdims=True)
  xn = xr / jnp.maximum(s, 1.0)
  h = jnp.dot(xn, w1_ref[...], preferred_element_type=jnp.float32)
  deg = degp_ref[0, :, 0:1] + degp_ref[1, :, 0:1] + 1.0
  dinv = lax.rsqrt(deg)
  hs_ref[...] = h * dinv


def _tc_b_body(aggp_ref, hs1_ref, degp_ref, b1_ref, w2_ref, hs2_ref):
  deg = degp_ref[0, :, 0:1] + degp_ref[1, :, 0:1] + 1.0
  dinv = lax.rsqrt(deg)
  t = dinv * (aggp_ref[0] + aggp_ref[1] + hs1_ref[...]) + b1_ref[...]
  t = jnp.maximum(t, 0.0)
  h2 = jnp.dot(t, w2_ref[...], preferred_element_type=jnp.float32)
  hs2_ref[...] = h2 * dinv


def _tc_c_body(aggp_ref, hs2_ref, degp_ref, b2_ref, batch_ref,
               fcw1_ref, fcb1_ref, fcw2_ref, fcb2_ref, out_ref, acc_ref):
  i = pl.program_id(0)

  @pl.when(i == 0)
  def _():
    acc_ref[...] = jnp.zeros((N_GRAPHS, HIDDEN), jnp.float32)

  deg = degp_ref[0, :, 0:1] + degp_ref[1, :, 0:1] + 1.0
  dinv = lax.rsqrt(deg)
  h2f = dinv * (aggp_ref[0] + aggp_ref[1] + hs2_ref[...]) + b2_ref[...]
  b = batch_ref[0, 0, :]
  onehot = (b[:, None] == lax.broadcasted_iota(jnp.int32, (1, N_GRAPHS), 1)
            ).astype(jnp.float32)
  gp = lax.dot_general(onehot, h2f, (((0,), (0,)), ((), ())),
                       preferred_element_type=jnp.float32)
  acc = acc_ref[...] + gp
  acc_ref[...] = acc

  @pl.when(i == N_BLKS - 1)
  def _():
    g1 = jnp.maximum(
        jnp.dot(acc, fcw1_ref[...], preferred_element_type=jnp.float32)
        + fcb1_ref[...], 0.0)
    logits = jnp.dot(g1, fcw2_ref[...],
                     preferred_element_type=jnp.float32) + fcb2_ref[...]
    m = jnp.max(logits, axis=1, keepdims=True)
    z = logits - m
    lse = jnp.log(jnp.sum(jnp.exp(z), axis=1, keepdims=True))
    out_ref[...] = z - lse


def _row_blk(i):
  return (i, 0)


def _full(i):
  return (0, 0)


def _part_blk(i):
  return (0, i, 0)


_tc_a = pl.pallas_call(
    _tc_a_body,
    grid=(N_BLKS,),
    in_specs=[
        pl.BlockSpec((ROW_BLK, D_FEAT), _row_blk),
        pl.BlockSpec((D_FEAT, HIDDEN), _full),
        pl.BlockSpec((2, ROW_BLK, DEG_W), _part_blk),
    ],
    out_specs=pl.BlockSpec((ROW_BLK, HIDDEN), _row_blk),
    out_shape=jax.ShapeDtypeStruct((N_NODES, HIDDEN), jnp.float32),
)

_tc_b = pl.pallas_call(
    _tc_b_body,
    grid=(N_BLKS,),
    in_specs=[
        pl.BlockSpec((2, ROW_BLK, HIDDEN), _part_blk),
        pl.BlockSpec((ROW_BLK, HIDDEN), _row_blk),
        pl.BlockSpec((2, ROW_BLK, DEG_W), _part_blk),
        pl.BlockSpec((1, HIDDEN), _full),
        pl.BlockSpec((HIDDEN, HIDDEN), _full),
    ],
    out_specs=pl.BlockSpec((ROW_BLK, HIDDEN), _row_blk),
    out_shape=jax.ShapeDtypeStruct((N_NODES, HIDDEN), jnp.float32),
)

_tc_c = pl.pallas_call(
    _tc_c_body,
    grid=(N_BLKS,),
    in_specs=[
        pl.BlockSpec((2, ROW_BLK, HIDDEN), _part_blk),
        pl.BlockSpec((ROW_BLK, HIDDEN), _row_blk),
        pl.BlockSpec((2, ROW_BLK, DEG_W), _part_blk),
        pl.BlockSpec((1, HIDDEN), _full),
        pl.BlockSpec((1, 1, ROW_BLK), lambda i: (i, 0, 0)),
        pl.BlockSpec((HIDDEN, HIDDEN), _full),
        pl.BlockSpec((1, HIDDEN), _full),
        pl.BlockSpec((HIDDEN, N_CLASSES), _full),
        pl.BlockSpec((1, N_CLASSES), _full),
    ],
    out_specs=pl.BlockSpec((N_GRAPHS, N_CLASSES), _full),
    out_shape=jax.ShapeDtypeStruct((N_GRAPHS, N_CLASSES), jnp.float32),
    scratch_shapes=[pltpu.VMEM((N_GRAPHS, HIDDEN), jnp.float32)],
)


@jax.jit
def kernel(x, edge_index, batch, W1, b1, W2, b2, fcW1, fcb1, fcW2, fcb2):
  src = edge_index[0].astype(jnp.int32)
  dst = edge_index[1].astype(jnp.int32)

  pad = EDGES_PER_TILE_PAD - EDGES_PER_TILE
  srcp = jnp.concatenate(
      [src.reshape(NW, EDGES_PER_TILE),
       jnp.zeros((NW, pad), jnp.int32)], axis=1).reshape(NW, K_CHUNKS, CHUNK)
  dstp = jnp.concatenate(
      [dst.reshape(NW, EDGES_PER_TILE),
       jnp.full((NW, pad), DUMMY_ROW, jnp.int32)],
      axis=1).reshape(NW, K_CHUNKS, CHUNK)

  sc_agg = _get_sc_agg()
  degp = _get_sc_deg()(dstp)

  hs1 = _tc_a(x, W1, degp)
  agg1 = sc_agg(hs1, srcp, dstp)
  hs2 = _tc_b(agg1, hs1, degp, b1.reshape(1, HIDDEN), W2)
  agg2 = sc_agg(hs2, srcp, dstp)
  out = _tc_c(agg2, hs2, degp, b2.reshape(1, HIDDEN),
              batch.astype(jnp.int32).reshape(N_BLKS, 1, ROW_BLK),
              fcW1, fcb1.reshape(1, HIDDEN), fcW2,
              fcb2.reshape(1, N_CLASSES))
  return out


# speedup vs baseline: 42.6251x; 1.6322x over previous
import functools

import jax
import jax.numpy as jnp
from jax import lax
from jax.experimental import pallas as pl
from jax.experimental.pallas import tpu as pltpu
from jax.experimental.pallas import tpu_sc as plsc

N_NODES = 10000
N_EDGES = 320000
D_FEAT = 128
HIDDEN = 32
N_CLASSES = 10
N_GRAPHS = 64

NC = 2
NS = 16
NW = NC * NS

EDGES_PER_TILE = N_EDGES // NW
CHUNK = 128
NBUF = 8
K_CHUNKS = 80
EDGES_PER_TILE_PAD = K_CHUNKS * CHUNK
DEG_W = 16
ROWS_PER_TILE = 632
N_ACC = ROWS_PER_TILE * NS
DUMMY_ROW = N_NODES

ROW_BLK = 1000
N_BLKS = N_NODES // ROW_BLK


def _zero_stripe(zbuf_v, acc_sh, sid, width):
  def zero_row(r, _):
    for c0 in range(0, width, 16):
      zbuf_v[r, pl.ds(c0, 16)] = jnp.zeros((16,), jnp.float32)
    return 0
  lax.fori_loop(0, ROWS_PER_TILE, zero_row, 0)
  pltpu.sync_copy(zbuf_v, acc_sh.at[pl.ds(sid * ROWS_PER_TILE, ROWS_PER_TILE)])


def _writeout(zbuf_v, acc_sh, out_hbm, cid, sid):
  pltpu.sync_copy(acc_sh.at[pl.ds(sid * ROWS_PER_TILE, ROWS_PER_TILE)],
                  zbuf_v)
  pltpu.sync_copy(zbuf_v,
                  out_hbm.at[cid, pl.ds(sid * ROWS_PER_TILE, ROWS_PER_TILE)])


def _sc_agg_kernel(table_hbm, src_hbm, dst_hbm, out_hbm,
                   src_v, dst_v, rows_v, zbuf_v, acc_sh, tbl_sh, sem_g, sem_s):
  cid = lax.axis_index("c")
  sid = lax.axis_index("s")
  wid = sid * NC + cid

  @pl.when(sid == 0)
  def _():
    pltpu.sync_copy(table_hbm, tbl_sh)

  _zero_stripe(zbuf_v, acc_sh, sid, HIDDEN)
  pltpu.sync_copy(src_hbm.at[wid], src_v)
  pltpu.sync_copy(dst_hbm.at[wid], dst_v)
  plsc.subcore_barrier()

  def group(g, _):
    j0 = g * NBUF
    gd = []
    for b in range(NBUF):
      @pl.when(g > 0)
      def _(b=b):
        pltpu.make_async_copy(rows_v.at[b], acc_sh.at[dst_v.at[j0 + b]],
                              sem_s).wait()
      gd.append(pltpu.async_copy(tbl_sh.at[src_v.at[j0 + b]],
                                 rows_v.at[b], sem_g))
    for b in range(NBUF):
      gd[b].wait()
      pltpu.async_copy(rows_v.at[b], acc_sh.at[dst_v.at[j0 + b]],
                       sem_s, add=True)
    return 0
  lax.fori_loop(0, K_CHUNKS // NBUF, group, 0)

  for b in range(NBUF):
    pltpu.make_async_copy(rows_v.at[b], acc_sh.at[dst_v.at[b]], sem_s).wait()

  plsc.subcore_barrier()
  _writeout(zbuf_v, acc_sh, out_hbm, cid, sid)


def _sc_deg_kernel(dst_hbm, out_hbm, dst_v, ones_v, zbuf_v, acc_sh, sem_s):
  cid = lax.axis_index("c")
  sid = lax.axis_index("s")
  wid = sid * NC + cid

  def one_row(r, _):
    ones_v[r, pl.ds(0, 16)] = jnp.ones((16,), jnp.float32)
    return 0
  lax.fori_loop(0, CHUNK, one_row, 0)
  _zero_stripe(zbuf_v, acc_sh, sid, DEG_W)
  pltpu.sync_copy(dst_hbm.at[wid], dst_v)
  plsc.subcore_barrier()

  def group(g, _):
    j0 = g * NBUF
    sd = [pltpu.async_copy(ones_v, acc_sh.at[dst_v.at[j0 + b]],
                           sem_s, add=True) for b in range(NBUF)]
    for d in sd:
      d.wait()
    return 0
  lax.fori_loop(0, K_CHUNKS // NBUF, group, 0)

  plsc.subcore_barrier()
  _writeout(zbuf_v, acc_sh, out_hbm, cid, sid)


@functools.cache
def _get_sc_agg():
  return pl.kernel(
      _sc_agg_kernel,
      out_type=jax.ShapeDtypeStruct((NC, N_ACC, HIDDEN), jnp.float32),
      mesh=plsc.VectorSubcoreMesh(core_axis_name="c", subcore_axis_name="s",
                                  num_cores=NC, num_subcores=NS),
      compiler_params=pltpu.CompilerParams(use_tc_tiling_on_sc=False),
      scratch_types=[
          pltpu.VMEM((K_CHUNKS, CHUNK), jnp.int32),
          pltpu.VMEM((K_CHUNKS, CHUNK), jnp.int32),
          pltpu.VMEM((NBUF, CHUNK, HIDDEN), jnp.float32),
          pltpu.VMEM((ROWS_PER_TILE, HIDDEN), jnp.float32),
          pltpu.VMEM_SHARED((N_ACC, HIDDEN), jnp.float32),
          pltpu.VMEM_SHARED((N_NODES, HIDDEN), jnp.float32),
          pltpu.SemaphoreType.DMA,
          pltpu.SemaphoreType.DMA,
      ],
  )


@functools.cache
def _get_sc_deg():
  return pl.kernel(
      _sc_deg_kernel,
      out_type=jax.ShapeDtypeStruct((NC, N_ACC, DEG_W), jnp.float32),
      mesh=plsc.VectorSubcoreMesh(core_axis_name="c", subcore_axis_name="s",
                                  num_cores=NC, num_subcores=NS),
      compiler_params=pltpu.CompilerParams(use_tc_tiling_on_sc=False),
      scratch_types=[
          pltpu.VMEM((K_CHUNKS, CHUNK), jnp.int32),
          pltpu.VMEM((CHUNK, DEG_W), jnp.float32),
          pltpu.VMEM((ROWS_PER_TILE, DEG_W), jnp.float32),
          pltpu.VMEM_SHARED((N_ACC, DEG_W), jnp.float32),
          pltpu.SemaphoreType.DMA,
      ],
  )


def _tc_a_body(x_ref, w1_ref, degp_ref, hs_ref):
  xr = x_ref[...]
  s = jnp.sum(xr, axis=1, keepdims=True)
  xn = xr / jnp.maximum(s, 1.0)
  h = jnp.dot(xn, w1_ref[...], preferred_element_type=jnp.float32)
  deg = degp_ref[0, :, 0:1] + degp_ref[1, :, 0:1] + 1.0
  dinv = lax.rsqrt(deg)
  hs_ref[...] = h * dinv


def _tc_b_body(aggp_ref, hs1_ref, degp_ref, b1_ref, w2_ref, hs2_ref):
  deg = degp_ref[0, :, 0:1] + degp_ref[1, :, 0:1] + 1.0
  dinv = lax.rsqrt(deg)
  t = dinv * (aggp_ref[0] + aggp_ref[1] + hs1_ref[...]) + b1_ref[...]
  t = jnp.maximum(t, 0.0)
  h2 = jnp.dot(t, w2_ref[...], preferred_element_type=jnp.float32)
  hs2_ref[...] = h2 * dinv


def _tc_c_body(aggp_ref, hs2_ref, degp_ref, b2_ref, batch_ref,
               fcw1_ref, fcb1_ref, fcw2_ref, fcb2_ref, out_ref, acc_ref):
  i = pl.program_id(0)

  @pl.when(i == 0)
  def _():
    acc_ref[...] = jnp.zeros((N_GRAPHS, HIDDEN), jnp.float32)

  deg = degp_ref[0, :, 0:1] + degp_ref[1, :, 0:1] + 1.0
  dinv = lax.rsqrt(deg)
  h2f = dinv * (aggp_ref[0] + aggp_ref[1] + hs2_ref[...]) + b2_ref[...]
  b = batch_ref[0, 0, :]
  onehot = (b[:, None] == lax.broadcasted_iota(jnp.int32, (1, N_GRAPHS), 1)
            ).astype(jnp.float32)
  gp = lax.dot_general(onehot, h2f, (((0,), (0,)), ((), ())),
                       preferred_element_type=jnp.float32)
  acc = acc_ref[...] + gp
  acc_ref[...] = acc

  @pl.when(i == N_BLKS - 1)
  def _():
    g1 = jnp.maximum(
        jnp.dot(acc, fcw1_ref[...], preferred_element_type=jnp.float32)
        + fcb1_ref[...], 0.0)
    logits = jnp.dot(g1, fcw2_ref[...],
                     preferred_element_type=jnp.float32) + fcb2_ref[...]
    m = jnp.max(logits, axis=1, keepdims=True)
    z = logits - m
    lse = jnp.log(jnp.sum(jnp.exp(z), axis=1, keepdims=True))
    out_ref[...] = z - lse


def _row_blk(i):
  return (i, 0)


def _full(i):
  return (0, 0)


def _part_blk(i):
  return (0, i, 0)


_tc_a = pl.pallas_call(
    _tc_a_body,
    grid=(N_BLKS,),
    in_specs=[
        pl.BlockSpec((ROW_BLK, D_FEAT), _row_blk),
        pl.BlockSpec((D_FEAT, HIDDEN), _full),
        pl.BlockSpec((2, ROW_BLK, DEG_W), _part_blk),
    ],
    out_specs=pl.BlockSpec((ROW_BLK, HIDDEN), _row_blk),
    out_shape=jax.ShapeDtypeStruct((N_NODES, HIDDEN), jnp.float32),
)

_tc_b = pl.pallas_call(
    _tc_b_body,
    grid=(N_BLKS,),
    in_specs=[
        pl.BlockSpec((2, ROW_BLK, HIDDEN), _part_blk),
        pl.BlockSpec((ROW_BLK, HIDDEN), _row_blk),
        pl.BlockSpec((2, ROW_BLK, DEG_W), _part_blk),
        pl.BlockSpec((1, HIDDEN), _full),
        pl.BlockSpec((HIDDEN, HIDDEN), _full),
    ],
    out_specs=pl.BlockSpec((ROW_BLK, HIDDEN), _row_blk),
    out_shape=jax.ShapeDtypeStruct((N_NODES, HIDDEN), jnp.float32),
)

_tc_c = pl.pallas_call(
    _tc_c_body,
    grid=(N_BLKS,),
    in_specs=[
        pl.BlockSpec((2, ROW_BLK, HIDDEN), _part_blk),
        pl.BlockSpec((ROW_BLK, HIDDEN), _row_blk),
        pl.BlockSpec((2, ROW_BLK, DEG_W), _part_blk),
        pl.BlockSpec((1, HIDDEN), _full),
        pl.BlockSpec((1, 1, ROW_BLK), lambda i: (i, 0, 0)),
        pl.BlockSpec((HIDDEN, HIDDEN), _full),
        pl.BlockSpec((1, HIDDEN), _full),
        pl.BlockSpec((HIDDEN, N_CLASSES), _full),
        pl.BlockSpec((1, N_CLASSES), _full),
    ],
    out_specs=pl.BlockSpec((N_GRAPHS, N_CLASSES), _full),
    out_shape=jax.ShapeDtypeStruct((N_GRAPHS, N_CLASSES), jnp.float32),
    scratch_shapes=[pltpu.VMEM((N_GRAPHS, HIDDEN), jnp.float32)],
)


@jax.jit
def kernel(x, edge_index, batch, W1, b1, W2, b2, fcW1, fcb1, fcW2, fcb2):
  src = edge_index[0].astype(jnp.int32)
  dst = edge_index[1].astype(jnp.int32)

  pad = EDGES_PER_TILE_PAD - EDGES_PER_TILE
  srcp = jnp.concatenate(
      [src.reshape(NW, EDGES_PER_TILE),
       jnp.zeros((NW, pad), jnp.int32)], axis=1).reshape(NW, K_CHUNKS, CHUNK)
  dstp = jnp.concatenate(
      [dst.reshape(NW, EDGES_PER_TILE),
       jnp.full((NW, pad), DUMMY_ROW, jnp.int32)],
      axis=1).reshape(NW, K_CHUNKS, CHUNK)

  sc_agg = _get_sc_agg()
  degp = _get_sc_deg()(dstp)

  hs1 = _tc_a(x, W1, degp)
  agg1 = sc_agg(hs1, srcp, dstp)
  hs2 = _tc_b(agg1, hs1, degp, b1.reshape(1, HIDDEN), W2)
  agg2 = sc_agg(hs2, srcp, dstp)
  out = _tc_c(agg2, hs2, degp, b2.reshape(1, HIDDEN),
              batch.astype(jnp.int32).reshape(N_BLKS, 1, ROW_BLK),
              fcW1, fcb1.reshape(1, HIDDEN), fcW2,
              fcb2.reshape(1, N_CLASSES))
  return out


# speedup vs baseline: 59.3746x; 1.3929x over previous
import functools

import jax
import jax.numpy as jnp
from jax import lax
from jax.experimental import pallas as pl
from jax.experimental.pallas import tpu as pltpu
from jax.experimental.pallas import tpu_sc as plsc

N_NODES = 10000
N_EDGES = 320000
D_FEAT = 128
HIDDEN = 32
N_CLASSES = 10
N_GRAPHS = 64

NC = 2
NS = 16
NW = NC * NS

EDGES_PER_TILE = N_EDGES // NW
CHUNK = 128
NBUF = 6
K_FULL = EDGES_PER_TILE // CHUNK
TAIL = EDGES_PER_TILE - K_FULL * CHUNK
DEG_W = 16
ROWS_PER_TILE = 632
N_ACC = ROWS_PER_TILE * NS

ROW_BLK = 2000
N_BLKS = N_NODES // ROW_BLK


def _zero_stripe(zbuf_v, acc_sh, sid, width):
  def zero_row(r, _):
    for c0 in range(0, width, 16):
      zbuf_v[r, pl.ds(c0, 16)] = jnp.zeros((16,), jnp.float32)
    return 0
  lax.fori_loop(0, ROWS_PER_TILE, zero_row, 0)
  pltpu.sync_copy(zbuf_v, acc_sh.at[pl.ds(sid * ROWS_PER_TILE, ROWS_PER_TILE)])


def _writeout(zbuf_v, acc_sh, out_hbm, cid, sid, width):
  pltpu.sync_copy(acc_sh.at[pl.ds(sid * ROWS_PER_TILE, ROWS_PER_TILE)],
                  zbuf_v)
  pltpu.sync_copy(zbuf_v,
                  out_hbm.at[cid, pl.ds(sid * ROWS_PER_TILE, ROWS_PER_TILE),
                             pl.ds(0, width)])


def _sc_agg_kernel(table_hbm, ei_hbm, out_hbm,
                   src_v, dst_v, rows_v, tail_v, zbuf_v, acc_sh, tbl_sh,
                   sem_g, sem_s):
  cid = lax.axis_index("c")
  sid = lax.axis_index("s")
  wid = sid * NC + cid
  base = wid * EDGES_PER_TILE

  @pl.when(sid == 0)
  def _():
    pltpu.sync_copy(table_hbm.at[pl.ds(0, N_NODES), pl.ds(0, HIDDEN)], tbl_sh)

  _zero_stripe(zbuf_v, acc_sh, sid, HIDDEN)
  pltpu.sync_copy(ei_hbm.at[0, pl.ds(base, EDGES_PER_TILE)], src_v)
  pltpu.sync_copy(ei_hbm.at[1, pl.ds(base, EDGES_PER_TILE)], dst_v)
  plsc.subcore_barrier()

  def g_idx(j):
    return src_v.at[pl.ds(j * CHUNK, CHUNK)]

  def s_idx(j):
    return dst_v.at[pl.ds(j * CHUNK, CHUNK)]

  def group(g, _):
    j0 = g * NBUF
    gd = []
    for b in range(NBUF):
      @pl.when(g > 0)
      def _(b=b):
        pltpu.make_async_copy(rows_v.at[b], acc_sh.at[s_idx(j0 + b)],
                              sem_s).wait()
      gd.append(pltpu.async_copy(tbl_sh.at[g_idx(j0 + b)],
                                 rows_v.at[b], sem_g))
    for b in range(NBUF):
      gd[b].wait()
      pltpu.async_copy(rows_v.at[b], acc_sh.at[s_idx(j0 + b)],
                       sem_s, add=True)
    return 0
  lax.fori_loop(0, K_FULL // NBUF, group, 0)

  pltpu.async_copy(
      tbl_sh.at[src_v.at[pl.ds(K_FULL * CHUNK, TAIL)]], tail_v, sem_g).wait()
  pltpu.sync_copy(tail_v, acc_sh.at[dst_v.at[pl.ds(K_FULL * CHUNK, TAIL)]],
                  add=True)
  for b in range(NBUF):
    pltpu.make_async_copy(rows_v.at[b], acc_sh.at[s_idx(b)], sem_s).wait()

  plsc.subcore_barrier()
  _writeout(zbuf_v, acc_sh, out_hbm, cid, sid, HIDDEN)


def _sc_deg_kernel(ei_hbm, out_hbm, dst_v, ones_v, zbuf_v, acc_sh, sem_s):
  cid = lax.axis_index("c")
  sid = lax.axis_index("s")
  wid = sid * NC + cid
  base = wid * EDGES_PER_TILE

  def one_row(r, _):
    ones_v[r, pl.ds(0, 16)] = jnp.ones((16,), jnp.float32)
    return 0
  lax.fori_loop(0, CHUNK, one_row, 0)
  _zero_stripe(zbuf_v, acc_sh, sid, DEG_W)
  pltpu.sync_copy(ei_hbm.at[1, pl.ds(base, EDGES_PER_TILE)], dst_v)
  plsc.subcore_barrier()

  def group(g, _):
    j0 = g * NBUF
    sd = [pltpu.async_copy(
        ones_v, acc_sh.at[dst_v.at[pl.ds((j0 + b) * CHUNK, CHUNK)]],
        sem_s, add=True) for b in range(NBUF)]
    for d in sd:
      d.wait()
    return 0
  lax.fori_loop(0, K_FULL // NBUF, group, 0)

  pltpu.sync_copy(ones_v.at[pl.ds(0, TAIL)],
                  acc_sh.at[dst_v.at[pl.ds(K_FULL * CHUNK, TAIL)]], add=True)

  plsc.subcore_barrier()
  _writeout(zbuf_v, acc_sh, out_hbm, cid, sid, DEG_W)


@functools.cache
def _get_sc_agg():
  return pl.kernel(
      _sc_agg_kernel,
      out_type=jax.ShapeDtypeStruct((NC, N_ACC, 128), jnp.float32),
      mesh=plsc.VectorSubcoreMesh(core_axis_name="c", subcore_axis_name="s",
                                  num_cores=NC, num_subcores=NS),
      compiler_params=pltpu.CompilerParams(use_tc_tiling_on_sc=False),
      scratch_types=[
          pltpu.VMEM((EDGES_PER_TILE,), jnp.int32),
          pltpu.VMEM((EDGES_PER_TILE,), jnp.int32),
          pltpu.VMEM((NBUF, CHUNK, HIDDEN), jnp.float32),
          pltpu.VMEM((TAIL, HIDDEN), jnp.float32),
          pltpu.VMEM((ROWS_PER_TILE, HIDDEN), jnp.float32),
          pltpu.VMEM_SHARED((N_ACC, HIDDEN), jnp.float32),
          pltpu.VMEM_SHARED((N_NODES, HIDDEN), jnp.float32),
          pltpu.SemaphoreType.DMA,
          pltpu.SemaphoreType.DMA,
      ],
  )


@functools.cache
def _get_sc_deg():
  return pl.kernel(
      _sc_deg_kernel,
      out_type=jax.ShapeDtypeStruct((NC, N_ACC, 128), jnp.float32),
      mesh=plsc.VectorSubcoreMesh(core_axis_name="c", subcore_axis_name="s",
                                  num_cores=NC, num_subcores=NS),
      compiler_params=pltpu.CompilerParams(use_tc_tiling_on_sc=False),
      scratch_types=[
          pltpu.VMEM((EDGES_PER_TILE,), jnp.int32),
          pltpu.VMEM((CHUNK, DEG_W), jnp.float32),
          pltpu.VMEM((ROWS_PER_TILE, DEG_W), jnp.float32),
          pltpu.VMEM_SHARED((N_ACC, DEG_W), jnp.float32),
          pltpu.SemaphoreType.DMA,
      ],
  )


def _tc_a1_body(x_ref, w1_ref, h_ref):
  xr = x_ref[...]
  s = jnp.sum(xr, axis=1, keepdims=True)
  xn = xr / jnp.maximum(s, 1.0)
  h_ref[...] = jnp.dot(xn, w1_ref[...], preferred_element_type=jnp.float32)


def _tc_a2_body(h_ref, degp_ref, hs_ref):
  deg = degp_ref[0, :, 0:1] + degp_ref[1, :, 0:1] + 1.0
  dinv = lax.rsqrt(deg)
  hs_ref[:, :HIDDEN] = h_ref[...] * dinv


def _tc_b_body(aggp_ref, hs1_ref, degp_ref, b1_ref, w2_ref, hs2_ref):
  deg = degp_ref[0, :, 0:1] + degp_ref[1, :, 0:1] + 1.0
  dinv = lax.rsqrt(deg)
  agg = aggp_ref[0, :, :HIDDEN] + aggp_ref[1, :, :HIDDEN]
  t = dinv * (agg + hs1_ref[:, :HIDDEN]) + b1_ref[...]
  t = jnp.maximum(t, 0.0)
  h2 = jnp.dot(t, w2_ref[...], preferred_element_type=jnp.float32)
  hs2_ref[:, :HIDDEN] = h2 * dinv


def _tc_c_body(aggp_ref, hs2_ref, degp_ref, b2_ref, batch_ref,
               fcw1_ref, fcb1_ref, fcw2_ref, fcb2_ref, out_ref, acc_ref):
  i = pl.program_id(0)

  @pl.when(i == 0)
  def _():
    acc_ref[...] = jnp.zeros((N_GRAPHS, HIDDEN), jnp.float32)

  deg = degp_ref[0, :, 0:1] + degp_ref[1, :, 0:1] + 1.0
  dinv = lax.rsqrt(deg)
  agg = aggp_ref[0, :, :HIDDEN] + aggp_ref[1, :, :HIDDEN]
  h2f = dinv * (agg + hs2_ref[:, :HIDDEN]) + b2_ref[...]
  b = batch_ref[0, 0, :]
  onehot = (b[:, None] == lax.broadcasted_iota(jnp.int32, (1, N_GRAPHS), 1)
            ).astype(jnp.float32)
  gp = lax.dot_general(onehot, h2f, (((0,), (0,)), ((), ())),
                       preferred_element_type=jnp.float32)
  acc = acc_ref[...] + gp
  acc_ref[...] = acc

  @pl.when(i == N_BLKS - 1)
  def _():
    g1 = jnp.maximum(
        jnp.dot(acc, fcw1_ref[...], preferred_element_type=jnp.float32)
        + fcb1_ref[...], 0.0)
    logits = jnp.dot(g1, fcw2_ref[...],
                     preferred_element_type=jnp.float32) + fcb2_ref[...]
    m = jnp.max(logits, axis=1, keepdims=True)
    z = logits - m
    lse = jnp.log(jnp.sum(jnp.exp(z), axis=1, keepdims=True))
    out_ref[...] = z - lse


def _row_blk(i):
  return (i, 0)


def _full(i):
  return (0, 0)


def _part_blk(i):
  return (0, i, 0)


_tc_a1 = pl.pallas_call(
    _tc_a1_body,
    grid=(N_BLKS,),
    in_specs=[
        pl.BlockSpec((ROW_BLK, D_FEAT), _row_blk),
        pl.BlockSpec((D_FEAT, HIDDEN), _full),
    ],
    out_specs=pl.BlockSpec((ROW_BLK, HIDDEN), _row_blk),
    out_shape=jax.ShapeDtypeStruct((N_NODES, HIDDEN), jnp.float32),
)

_tc_a2 = pl.pallas_call(
    _tc_a2_body,
    grid=(N_BLKS,),
    in_specs=[
        pl.BlockSpec((ROW_BLK, HIDDEN), _row_blk),
        pl.BlockSpec((2, ROW_BLK, 128), _part_blk),
    ],
    out_specs=pl.BlockSpec((ROW_BLK, 128), _row_blk),
    out_shape=jax.ShapeDtypeStruct((N_ACC, 128), jnp.float32),
)

_tc_b = pl.pallas_call(
    _tc_b_body,
    grid=(N_BLKS,),
    in_specs=[
        pl.BlockSpec((2, ROW_BLK, 128), _part_blk),
        pl.BlockSpec((ROW_BLK, 128), _row_blk),
        pl.BlockSpec((2, ROW_BLK, 128), _part_blk),
        pl.BlockSpec((1, HIDDEN), _full),
        pl.BlockSpec((HIDDEN, HIDDEN), _full),
    ],
    out_specs=pl.BlockSpec((ROW_BLK, 128), _row_blk),
    out_shape=jax.ShapeDtypeStruct((N_ACC, 128), jnp.float32),
)

_tc_c = pl.pallas_call(
    _tc_c_body,
    grid=(N_BLKS,),
    in_specs=[
        pl.BlockSpec((2, ROW_BLK, 128), _part_blk),
        pl.BlockSpec((ROW_BLK, 128), _row_blk),
        pl.BlockSpec((2, ROW_BLK, 128), _part_blk),
        pl.BlockSpec((1, HIDDEN), _full),
        pl.BlockSpec((1, 1, ROW_BLK), lambda i: (i, 0, 0)),
        pl.BlockSpec((HIDDEN, HIDDEN), _full),
        pl.BlockSpec((1, HIDDEN), _full),
        pl.BlockSpec((HIDDEN, N_CLASSES), _full),
        pl.BlockSpec((1, N_CLASSES), _full),
    ],
    out_specs=pl.BlockSpec((N_GRAPHS, N_CLASSES), _full),
    out_shape=jax.ShapeDtypeStruct((N_GRAPHS, N_CLASSES), jnp.float32),
    scratch_shapes=[pltpu.VMEM((N_GRAPHS, HIDDEN), jnp.float32)],
)


@jax.jit
def kernel(x, edge_index, batch, W1, b1, W2, b2, fcW1, fcb1, fcW2, fcb2):
  ei = edge_index.astype(jnp.int32)

  sc_agg = _get_sc_agg()
  degp = _get_sc_deg()(ei)

  h1 = _tc_a1(x, W1)
  hs1 = _tc_a2(h1, degp)
  agg1 = sc_agg(hs1, ei)
  hs2 = _tc_b(agg1, hs1, degp, b1.reshape(1, HIDDEN), W2)
  agg2 = sc_agg(hs2, ei)

  out = _tc_c(agg2, hs2, degp, b2.reshape(1, HIDDEN),
              batch.astype(jnp.int32).reshape(N_BLKS, 1, ROW_BLK),
              fcW1, fcb1.reshape(1, HIDDEN), fcW2,
              fcb2.reshape(1, N_CLASSES))
  return out


# speedup vs baseline: 60.6977x; 1.0223x over previous
import functools

import jax
import jax.numpy as jnp
from jax import lax
from jax.experimental import pallas as pl
from jax.experimental.pallas import tpu as pltpu
from jax.experimental.pallas import tpu_sc as plsc

N_NODES = 10000
N_EDGES = 320000
D_FEAT = 128
HIDDEN = 32
N_CLASSES = 10
N_GRAPHS = 64

NC = 2
NS = 16
NW = NC * NS

EDGES_PER_TILE = N_EDGES // NW
CHUNK = 128
NBUF = 6
K_FULL = EDGES_PER_TILE // CHUNK
TAIL = EDGES_PER_TILE - K_FULL * CHUNK
DEG_W = 16
ROWS_PER_TILE = 632
N_ACC = ROWS_PER_TILE * NS

ROW_BLK = 2000
N_BLKS = N_NODES // ROW_BLK


def _zero_stripe(zbuf_v, acc_sh, sid, width):
  def zero_row(r, _):
    for c0 in range(0, width, 16):
      zbuf_v[r, pl.ds(c0, 16)] = jnp.zeros((16,), jnp.float32)
    return 0
  lax.fori_loop(0, ROWS_PER_TILE, zero_row, 0)
  pltpu.sync_copy(zbuf_v, acc_sh.at[pl.ds(sid * ROWS_PER_TILE, ROWS_PER_TILE)])


def _writeout(zbuf_v, acc_sh, out_hbm, cid, sid, width):
  pltpu.sync_copy(acc_sh.at[pl.ds(sid * ROWS_PER_TILE, ROWS_PER_TILE)],
                  zbuf_v)
  pltpu.sync_copy(zbuf_v,
                  out_hbm.at[cid, pl.ds(sid * ROWS_PER_TILE, ROWS_PER_TILE),
                             pl.ds(0, width)])


def _sc_agg_kernel(table_hbm, ei_hbm, out_hbm,
                   src_v, dst_v, rows_v, tail_v, zbuf_v, acc_sh, tbl_sh,
                   sem_g, sem_s):
  cid = lax.axis_index("c")
  sid = lax.axis_index("s")
  wid = sid * NC + cid
  base = wid * EDGES_PER_TILE

  r0 = sid * ROWS_PER_TILE
  pltpu.sync_copy(
      table_hbm.at[pl.ds(r0, ROWS_PER_TILE), pl.ds(0, HIDDEN)],
      tbl_sh.at[pl.ds(r0, ROWS_PER_TILE)])

  _zero_stripe(zbuf_v, acc_sh, sid, HIDDEN)
  pltpu.sync_copy(ei_hbm.at[0, pl.ds(base, EDGES_PER_TILE)], src_v)
  pltpu.sync_copy(ei_hbm.at[1, pl.ds(base, EDGES_PER_TILE)], dst_v)
  plsc.subcore_barrier()

  def g_idx(j):
    return src_v.at[pl.ds(j * CHUNK, CHUNK)]

  def s_idx(j):
    return dst_v.at[pl.ds(j * CHUNK, CHUNK)]

  def group(g, _):
    j0 = g * NBUF
    gd = []
    for b in range(NBUF):
      @pl.when(g > 0)
      def _(b=b):
        pltpu.make_async_copy(rows_v.at[b], acc_sh.at[s_idx(j0 + b)],
                              sem_s).wait()
      gd.append(pltpu.async_copy(tbl_sh.at[g_idx(j0 + b)],
                                 rows_v.at[b], sem_g))
    for b in range(NBUF):
      gd[b].wait()
      pltpu.async_copy(rows_v.at[b], acc_sh.at[s_idx(j0 + b)],
                       sem_s, add=True)
    return 0
  lax.fori_loop(0, K_FULL // NBUF, group, 0)

  pltpu.async_copy(
      tbl_sh.at[src_v.at[pl.ds(K_FULL * CHUNK, TAIL)]], tail_v, sem_g).wait()
  pltpu.sync_copy(tail_v, acc_sh.at[dst_v.at[pl.ds(K_FULL * CHUNK, TAIL)]],
                  add=True)
  for b in range(NBUF):
    pltpu.make_async_copy(rows_v.at[b], acc_sh.at[s_idx(b)], sem_s).wait()

  plsc.subcore_barrier()
  _writeout(zbuf_v, acc_sh, out_hbm, cid, sid, HIDDEN)


def _sc_deg_kernel(ei_hbm, out_hbm, dst_v, ones_v, zbuf_v, acc_sh, sem_s):
  cid = lax.axis_index("c")
  sid = lax.axis_index("s")
  wid = sid * NC + cid
  base = wid * EDGES_PER_TILE

  def one_row(r, _):
    ones_v[r, pl.ds(0, 16)] = jnp.ones((16,), jnp.float32)
    return 0
  lax.fori_loop(0, CHUNK, one_row, 0)
  _zero_stripe(zbuf_v, acc_sh, sid, DEG_W)
  pltpu.sync_copy(ei_hbm.at[1, pl.ds(base, EDGES_PER_TILE)], dst_v)
  plsc.subcore_barrier()

  def group(g, _):
    j0 = g * NBUF
    sd = [pltpu.async_copy(
        ones_v, acc_sh.at[dst_v.at[pl.ds((j0 + b) * CHUNK, CHUNK)]],
        sem_s, add=True) for b in range(NBUF)]
    for d in sd:
      d.wait()
    return 0
  lax.fori_loop(0, K_FULL // NBUF, group, 0)

  pltpu.sync_copy(ones_v.at[pl.ds(0, TAIL)],
                  acc_sh.at[dst_v.at[pl.ds(K_FULL * CHUNK, TAIL)]], add=True)

  plsc.subcore_barrier()
  _writeout(zbuf_v, acc_sh, out_hbm, cid, sid, DEG_W)


@functools.cache
def _get_sc_agg():
  return pl.kernel(
      _sc_agg_kernel,
      out_type=jax.ShapeDtypeStruct((NC, N_ACC, 128), jnp.float32),
      mesh=plsc.VectorSubcoreMesh(core_axis_name="c", subcore_axis_name="s",
                                  num_cores=NC, num_subcores=NS),
      compiler_params=pltpu.CompilerParams(use_tc_tiling_on_sc=False),
      scratch_types=[
          pltpu.VMEM((EDGES_PER_TILE,), jnp.int32),
          pltpu.VMEM((EDGES_PER_TILE,), jnp.int32),
          pltpu.VMEM((NBUF, CHUNK, HIDDEN), jnp.float32),
          pltpu.VMEM((TAIL, HIDDEN), jnp.float32),
          pltpu.VMEM((ROWS_PER_TILE, HIDDEN), jnp.float32),
          pltpu.VMEM_SHARED((N_ACC, HIDDEN), jnp.float32),
          pltpu.VMEM_SHARED((N_ACC, HIDDEN), jnp.float32),
          pltpu.SemaphoreType.DMA,
          pltpu.SemaphoreType.DMA,
      ],
  )


@functools.cache
def _get_sc_deg():
  return pl.kernel(
      _sc_deg_kernel,
      out_type=jax.ShapeDtypeStruct((NC, N_ACC, 128), jnp.float32),
      mesh=plsc.VectorSubcoreMesh(core_axis_name="c", subcore_axis_name="s",
                                  num_cores=NC, num_subcores=NS),
      compiler_params=pltpu.CompilerParams(use_tc_tiling_on_sc=False),
      scratch_types=[
          pltpu.VMEM((EDGES_PER_TILE,), jnp.int32),
          pltpu.VMEM((CHUNK, DEG_W), jnp.float32),
          pltpu.VMEM((ROWS_PER_TILE, DEG_W), jnp.float32),
          pltpu.VMEM_SHARED((N_ACC, DEG_W), jnp.float32),
          pltpu.SemaphoreType.DMA,
      ],
  )


def _tc_a1_body(x_ref, w1_ref, h_ref):
  xr = x_ref[...]
  s = jnp.sum(xr, axis=1, keepdims=True)
  xn = xr / jnp.maximum(s, 1.0)
  h_ref[...] = jnp.dot(xn, w1_ref[...], preferred_element_type=jnp.float32)


def _tc_a2_body(h_ref, degp_ref, hs_ref):
  deg = degp_ref[0, :, 0:1] + degp_ref[1, :, 0:1] + 1.0
  dinv = lax.rsqrt(deg)
  hs_ref[:, :HIDDEN] = h_ref[...] * dinv
  hs_ref[:, HIDDEN:HIDDEN + 1] = dinv


def _tc_b_body(aggp_ref, hs1_ref, b1_ref, w2_ref, hs2_ref):
  dinv = hs1_ref[:, HIDDEN:HIDDEN + 1]
  agg = aggp_ref[0, :, :HIDDEN] + aggp_ref[1, :, :HIDDEN]
  t = dinv * (agg + hs1_ref[:, :HIDDEN]) + b1_ref[...]
  t = jnp.maximum(t, 0.0)
  h2 = jnp.dot(t, w2_ref[...], preferred_element_type=jnp.float32)
  hs2_ref[:, :HIDDEN] = h2 * dinv
  hs2_ref[:, HIDDEN:HIDDEN + 1] = dinv


def _tc_c_body(aggp_ref, hs2_ref, b2_ref, batch_ref,
               fcw1_ref, fcb1_ref, fcw2_ref, fcb2_ref, out_ref, acc_ref):
  i = pl.program_id(0)

  @pl.when(i == 0)
  def _():
    acc_ref[...] = jnp.zeros((N_GRAPHS, HIDDEN), jnp.float32)

  dinv = hs2_ref[:, HIDDEN:HIDDEN + 1]
  agg = aggp_ref[0, :, :HIDDEN] + aggp_ref[1, :, :HIDDEN]
  h2f = dinv * (agg + hs2_ref[:, :HIDDEN]) + b2_ref[...]
  b = batch_ref[0, 0, :]
  onehot = (b[:, None] == lax.broadcasted_iota(jnp.int32, (1, N_GRAPHS), 1)
            ).astype(jnp.float32)
  gp = lax.dot_general(onehot, h2f, (((0,), (0,)), ((), ())),
                       preferred_element_type=jnp.float32)
  acc = acc_ref[...] + gp
  acc_ref[...] = acc

  @pl.when(i == N_BLKS - 1)
  def _():
    g1 = jnp.maximum(
        jnp.dot(acc, fcw1_ref[...], preferred_element_type=jnp.float32)
        + fcb1_ref[...], 0.0)
    logits = jnp.dot(g1, fcw2_ref[...],
                     preferred_element_type=jnp.float32) + fcb2_ref[...]
    m = jnp.max(logits, axis=1, keepdims=True)
    z = logits - m
    lse = jnp.log(jnp.sum(jnp.exp(z), axis=1, keepdims=True))
    out_ref[...] = z - lse


def _row_blk(i):
  return (i, 0)


def _full(i):
  return (0, 0)


def _part_blk(i):
  return (0, i, 0)


_tc_a1 = pl.pallas_call(
    _tc_a1_body,
    grid=(N_BLKS,),
    in_specs=[
        pl.BlockSpec((ROW_BLK, D_FEAT), _row_blk),
        pl.BlockSpec((D_FEAT, HIDDEN), _full),
    ],
    out_specs=pl.BlockSpec((ROW_BLK, HIDDEN), _row_blk),
    out_shape=jax.ShapeDtypeStruct((N_NODES, HIDDEN), jnp.float32),
)

_tc_a2 = pl.pallas_call(
    _tc_a2_body,
    grid=(N_BLKS,),
    in_specs=[
        pl.BlockSpec((ROW_BLK, HIDDEN), _row_blk),
        pl.BlockSpec((2, ROW_BLK, 128), _part_blk),
    ],
    out_specs=pl.BlockSpec((ROW_BLK, 128), _row_blk),
    out_shape=jax.ShapeDtypeStruct((N_ACC, 128), jnp.float32),
)

_tc_b = pl.pallas_call(
    _tc_b_body,
    grid=(N_BLKS,),
    in_specs=[
        pl.BlockSpec((2, ROW_BLK, 128), _part_blk),
        pl.BlockSpec((ROW_BLK, 128), _row_blk),
        pl.BlockSpec((1, HIDDEN), _full),
        pl.BlockSpec((HIDDEN, HIDDEN), _full),
    ],
    out_specs=pl.BlockSpec((ROW_BLK, 128), _row_blk),
    out_shape=jax.ShapeDtypeStruct((N_ACC, 128), jnp.float32),
)

_tc_c = pl.pallas_call(
    _tc_c_body,
    grid=(N_BLKS,),
    in_specs=[
        pl.BlockSpec((2, ROW_BLK, 128), _part_blk),
        pl.BlockSpec((ROW_BLK, 128), _row_blk),
        pl.BlockSpec((1, HIDDEN), _full),
        pl.BlockSpec((1, 1, ROW_BLK), lambda i: (i, 0, 0)),
        pl.BlockSpec((HIDDEN, HIDDEN), _full),
        pl.BlockSpec((1, HIDDEN), _full),
        pl.BlockSpec((HIDDEN, N_CLASSES), _full),
        pl.BlockSpec((1, N_CLASSES), _full),
    ],
    out_specs=pl.BlockSpec((N_GRAPHS, N_CLASSES), _full),
    out_shape=jax.ShapeDtypeStruct((N_GRAPHS, N_CLASSES), jnp.float32),
    scratch_shapes=[pltpu.VMEM((N_GRAPHS, HIDDEN), jnp.float32)],
)


@jax.jit
def kernel(x, edge_index, batch, W1, b1, W2, b2, fcW1, fcb1, fcW2, fcb2):
  ei = edge_index.astype(jnp.int32)

  sc_agg = _get_sc_agg()
  degp = _get_sc_deg()(ei)

  h1 = _tc_a1(x, W1)
  hs1 = _tc_a2(h1, degp)
  agg1 = sc_agg(hs1, ei)
  hs2 = _tc_b(agg1, hs1, b1.reshape(1, HIDDEN), W2)
  agg2 = sc_agg(hs2, ei)

  out = _tc_c(agg2, hs2, b2.reshape(1, HIDDEN),
              batch.astype(jnp.int32).reshape(N_BLKS, 1, ROW_BLK),
              fcW1, fcb1.reshape(1, HIDDEN), fcW2,
              fcb2.reshape(1, N_CLASSES))
  return out


# speedup vs baseline: 64.1896x; 1.0575x over previous
import functools

import jax
import jax.numpy as jnp
from jax import lax
from jax.experimental import pallas as pl
from jax.experimental.pallas import tpu as pltpu
from jax.experimental.pallas import tpu_sc as plsc

N_NODES = 10000
N_EDGES = 320000
D_FEAT = 128
HIDDEN = 32
N_CLASSES = 10
N_GRAPHS = 64

NC = 2
NS = 16
NW = NC * NS

EDGES_PER_TILE = N_EDGES // NW
CHUNK = 128
NBUF = 6
K_FULL = EDGES_PER_TILE // CHUNK
TAIL = EDGES_PER_TILE - K_FULL * CHUNK
DEG_W = 16
ROWS_PER_TILE = 632
N_ACC = ROWS_PER_TILE * NS

ROW_BLK = 2000
N_BLKS = N_NODES // ROW_BLK


ZCHUNK = ROWS_PER_TILE // 4


def _zero_stripe(zbuf_v, acc_sh, sid, width):
  def zero_row(r, _):
    for c0 in range(0, width, 16):
      zbuf_v[r, pl.ds(c0, 16)] = jnp.zeros((16,), jnp.float32)
    return 0
  lax.fori_loop(0, ZCHUNK, zero_row, 0, unroll=8)
  for q in range(4):
    pltpu.sync_copy(zbuf_v.at[pl.ds(0, ZCHUNK)],
                    acc_sh.at[pl.ds(sid * ROWS_PER_TILE + q * ZCHUNK, ZCHUNK)])


def _writeout(zbuf_v, acc_sh, out_hbm, cid, sid, width):
  pltpu.sync_copy(acc_sh.at[pl.ds(sid * ROWS_PER_TILE, ROWS_PER_TILE)],
                  zbuf_v)
  pltpu.sync_copy(zbuf_v,
                  out_hbm.at[pl.ds(sid * ROWS_PER_TILE, ROWS_PER_TILE),
                             pl.ds(cid * width, width)])


def _sc_agg_kernel(table_hbm, ei_hbm, out_hbm,
                   src_v, dst_v, rows_v, tail_v, zbuf_v, acc_sh, tbl_sh,
                   sem_g, sem_s):
  cid = lax.axis_index("c")
  sid = lax.axis_index("s")
  wid = sid * NC + cid
  base = wid * EDGES_PER_TILE

  r0 = sid * ROWS_PER_TILE
  pltpu.sync_copy(
      table_hbm.at[pl.ds(r0, ROWS_PER_TILE), pl.ds(0, HIDDEN)],
      tbl_sh.at[pl.ds(r0, ROWS_PER_TILE)])

  _zero_stripe(zbuf_v, acc_sh, sid, HIDDEN)
  pltpu.sync_copy(ei_hbm.at[0, pl.ds(base, EDGES_PER_TILE)], src_v)
  pltpu.sync_copy(ei_hbm.at[1, pl.ds(base, EDGES_PER_TILE)], dst_v)
  plsc.subcore_barrier()

  def g_idx(j):
    return src_v.at[pl.ds(j * CHUNK, CHUNK)]

  def s_idx(j):
    return dst_v.at[pl.ds(j * CHUNK, CHUNK)]

  def group(g, _):
    j0 = g * NBUF
    gd = []
    for b in range(NBUF):
      @pl.when(g > 0)
      def _(b=b):
        pltpu.make_async_copy(rows_v.at[b], acc_sh.at[s_idx(j0 + b)],
                              sem_s).wait()
      gd.append(pltpu.async_copy(tbl_sh.at[g_idx(j0 + b)],
                                 rows_v.at[b], sem_g))
    for b in range(NBUF):
      gd[b].wait()
      pltpu.async_copy(rows_v.at[b], acc_sh.at[s_idx(j0 + b)],
                       sem_s, add=True)
    return 0
  lax.fori_loop(0, K_FULL // NBUF, group, 0)

  pltpu.async_copy(
      tbl_sh.at[src_v.at[pl.ds(K_FULL * CHUNK, TAIL)]], tail_v, sem_g).wait()
  pltpu.sync_copy(tail_v, acc_sh.at[dst_v.at[pl.ds(K_FULL * CHUNK, TAIL)]],
                  add=True)
  for b in range(NBUF):
    pltpu.make_async_copy(rows_v.at[b], acc_sh.at[s_idx(b)], sem_s).wait()

  plsc.subcore_barrier()
  _writeout(zbuf_v, acc_sh, out_hbm, cid, sid, HIDDEN)


def _sc_deg_kernel(ei_hbm, out_hbm, dst_v, ones_v, zbuf_v, acc_sh, sem_s):
  cid = lax.axis_index("c")
  sid = lax.axis_index("s")
  wid = sid * NC + cid
  base = wid * EDGES_PER_TILE

  def one_row(r, _):
    ones_v[r, pl.ds(0, 16)] = jnp.ones((16,), jnp.float32)
    return 0
  lax.fori_loop(0, CHUNK, one_row, 0)
  _zero_stripe(zbuf_v, acc_sh, sid, DEG_W)
  pltpu.sync_copy(ei_hbm.at[1, pl.ds(base, EDGES_PER_TILE)], dst_v)
  plsc.subcore_barrier()

  def group(g, _):
    j0 = g * NBUF
    sd = [pltpu.async_copy(
        ones_v, acc_sh.at[dst_v.at[pl.ds((j0 + b) * CHUNK, CHUNK)]],
        sem_s, add=True) for b in range(NBUF)]
    for d in sd:
      d.wait()
    return 0
  lax.fori_loop(0, K_FULL // NBUF, group, 0)

  pltpu.sync_copy(ones_v.at[pl.ds(0, TAIL)],
                  acc_sh.at[dst_v.at[pl.ds(K_FULL * CHUNK, TAIL)]], add=True)

  plsc.subcore_barrier()
  _writeout(zbuf_v, acc_sh, out_hbm, cid, sid, DEG_W)


@functools.cache
def _get_sc_agg():
  return pl.kernel(
      _sc_agg_kernel,
      out_type=jax.ShapeDtypeStruct((N_ACC, 128), jnp.float32),
      mesh=plsc.VectorSubcoreMesh(core_axis_name="c", subcore_axis_name="s",
                                  num_cores=NC, num_subcores=NS),
      compiler_params=pltpu.CompilerParams(use_tc_tiling_on_sc=False),
      scratch_types=[
          pltpu.VMEM((EDGES_PER_TILE,), jnp.int32),
          pltpu.VMEM((EDGES_PER_TILE,), jnp.int32),
          pltpu.VMEM((NBUF, CHUNK, HIDDEN), jnp.float32),
          pltpu.VMEM((TAIL, HIDDEN), jnp.float32),
          pltpu.VMEM((ROWS_PER_TILE, HIDDEN), jnp.float32),
          pltpu.VMEM_SHARED((N_ACC, HIDDEN), jnp.float32),
          pltpu.VMEM_SHARED((N_ACC, HIDDEN), jnp.float32),
          pltpu.SemaphoreType.DMA,
          pltpu.SemaphoreType.DMA,
      ],
  )


@functools.cache
def _get_sc_deg():
  return pl.kernel(
      _sc_deg_kernel,
      out_type=jax.ShapeDtypeStruct((N_ACC, 128), jnp.float32),
      mesh=plsc.VectorSubcoreMesh(core_axis_name="c", subcore_axis_name="s",
                                  num_cores=NC, num_subcores=NS),
      compiler_params=pltpu.CompilerParams(use_tc_tiling_on_sc=False),
      scratch_types=[
          pltpu.VMEM((EDGES_PER_TILE,), jnp.int32),
          pltpu.VMEM((CHUNK, DEG_W), jnp.float32),
          pltpu.VMEM((ROWS_PER_TILE, DEG_W), jnp.float32),
          pltpu.VMEM_SHARED((N_ACC, DEG_W), jnp.float32),
          pltpu.SemaphoreType.DMA,
      ],
  )


def _tc_a1_body(x_ref, w1_ref, h_ref):
  xr = x_ref[...]
  s = jnp.sum(xr, axis=1, keepdims=True)
  xn = xr / jnp.maximum(s, 1.0)
  h_ref[...] = jnp.dot(xn, w1_ref[...], preferred_element_type=jnp.float32)


def _tc_a2_body(h_ref, degp_ref, hs_ref):
  deg = degp_ref[:, 0:1] + degp_ref[:, DEG_W:DEG_W + 1] + 1.0
  dinv = lax.rsqrt(deg)
  hs_ref[:, :HIDDEN] = h_ref[...] * dinv
  hs_ref[:, HIDDEN:HIDDEN + 1] = dinv


def _tc_b_body(aggp_ref, hs1_ref, b1_ref, w2_ref, hs2_ref):
  dinv = hs1_ref[:, HIDDEN:HIDDEN + 1]
  agg = aggp_ref[:, :HIDDEN] + aggp_ref[:, HIDDEN:2 * HIDDEN]
  t = dinv * (agg + hs1_ref[:, :HIDDEN]) + b1_ref[...]
  t = jnp.maximum(t, 0.0)
  h2 = jnp.dot(t, w2_ref[...], preferred_element_type=jnp.float32)
  hs2_ref[:, :HIDDEN] = h2 * dinv
  hs2_ref[:, HIDDEN:HIDDEN + 1] = dinv


def _tc_c_body(aggp_ref, hs2_ref, b2_ref, batch_ref,
               fcw1_ref, fcb1_ref, fcw2_ref, fcb2_ref, out_ref, acc_ref):
  i = pl.program_id(0)

  @pl.when(i == 0)
  def _():
    acc_ref[...] = jnp.zeros((N_GRAPHS, HIDDEN), jnp.float32)

  dinv = hs2_ref[:, HIDDEN:HIDDEN + 1]
  agg = aggp_ref[:, :HIDDEN] + aggp_ref[:, HIDDEN:2 * HIDDEN]
  h2f = dinv * (agg + hs2_ref[:, :HIDDEN]) + b2_ref[...]
  b = batch_ref[0, 0, :]
  onehot = (b[:, None] == lax.broadcasted_iota(jnp.int32, (1, N_GRAPHS), 1)
            ).astype(jnp.float32)
  gp = lax.dot_general(onehot, h2f, (((0,), (0,)), ((), ())),
                       preferred_element_type=jnp.float32)
  acc = acc_ref[...] + gp
  acc_ref[...] = acc

  @pl.when(i == N_BLKS - 1)
  def _():
    g1 = jnp.maximum(
        jnp.dot(acc, fcw1_ref[...], preferred_element_type=jnp.float32)
        + fcb1_ref[...], 0.0)
    logits = jnp.dot(g1, fcw2_ref[...],
                     preferred_element_type=jnp.float32) + fcb2_ref[...]
    m = jnp.max(logits, axis=1, keepdims=True)
    z = logits - m
    lse = jnp.log(jnp.sum(jnp.exp(z), axis=1, keepdims=True))
    out_ref[...] = z - lse


def _row_blk(i):
  return (i, 0)


def _full(i):
  return (0, 0)


def _part_blk(i):
  return (0, i, 0)


_tc_a1 = pl.pallas_call(
    _tc_a1_body,
    grid=(N_BLKS,),
    in_specs=[
        pl.BlockSpec((ROW_BLK, D_FEAT), _row_blk),
        pl.BlockSpec((D_FEAT, HIDDEN), _full),
    ],
    out_specs=pl.BlockSpec((ROW_BLK, HIDDEN), _row_blk),
    out_shape=jax.ShapeDtypeStruct((N_NODES, HIDDEN), jnp.float32),
)

_tc_a2 = pl.pallas_call(
    _tc_a2_body,
    grid=(N_BLKS,),
    in_specs=[
        pl.BlockSpec((ROW_BLK, HIDDEN), _row_blk),
        pl.BlockSpec((ROW_BLK, 128), _row_blk),
    ],
    out_specs=pl.BlockSpec((ROW_BLK, 128), _row_blk),
    out_shape=jax.ShapeDtypeStruct((N_ACC, 128), jnp.float32),
)

_tc_b = pl.pallas_call(
    _tc_b_body,
    grid=(N_BLKS,),
    in_specs=[
        pl.BlockSpec((ROW_BLK, 128), _row_blk),
        pl.BlockSpec((ROW_BLK, 128), _row_blk),
        pl.BlockSpec((1, HIDDEN), _full),
        pl.BlockSpec((HIDDEN, HIDDEN), _full),
    ],
    out_specs=pl.BlockSpec((ROW_BLK, 128), _row_blk),
    out_shape=jax.ShapeDtypeStruct((N_ACC, 128), jnp.float32),
)

_tc_c = pl.pallas_call(
    _tc_c_body,
    grid=(N_BLKS,),
    in_specs=[
        pl.BlockSpec((ROW_BLK, 128), _row_blk),
        pl.BlockSpec((ROW_BLK, 128), _row_blk),
        pl.BlockSpec((1, HIDDEN), _full),
        pl.BlockSpec((1, 1, ROW_BLK), lambda i: (i, 0, 0)),
        pl.BlockSpec((HIDDEN, HIDDEN), _full),
        pl.BlockSpec((1, HIDDEN), _full),
        pl.BlockSpec((HIDDEN, N_CLASSES), _full),
        pl.BlockSpec((1, N_CLASSES), _full),
    ],
    out_specs=pl.BlockSpec((N_GRAPHS, N_CLASSES), _full),
    out_shape=jax.ShapeDtypeStruct((N_GRAPHS, N_CLASSES), jnp.float32),
    scratch_shapes=[pltpu.VMEM((N_GRAPHS, HIDDEN), jnp.float32)],
)


@jax.jit
def kernel(x, edge_index, batch, W1, b1, W2, b2, fcW1, fcb1, fcW2, fcb2):
  ei = edge_index.astype(jnp.int32)

  sc_agg = _get_sc_agg()
  degp = _get_sc_deg()(ei)

  h1 = _tc_a1(x, W1)
  hs1 = _tc_a2(h1, degp)
  agg1 = sc_agg(hs1, ei)
  hs2 = _tc_b(agg1, hs1, b1.reshape(1, HIDDEN), W2)
  agg2 = sc_agg(hs2, ei)

  out = _tc_c(agg2, hs2, b2.reshape(1, HIDDEN),
              batch.astype(jnp.int32).reshape(N_BLKS, 1, ROW_BLK),
              fcW1, fcb1.reshape(1, HIDDEN), fcW2,
              fcb2.reshape(1, N_CLASSES))
  return out
